# baseline (speedup 1.0000x reference)
import functools

import jax
import jax.numpy as jnp
from jax import lax
from jax.experimental import pallas as pl
from jax.experimental.pallas import tpu as pltpu
from jax.experimental.pallas import tpu_sc as plsc

N = 10000
D = 128
E = 320000

NC = 2
NS = 16

HALF = 5120
ACC_ROWS = HALF + 8
OUT_PER_TILE = HALF // NS
ZROWS = 160

CHUNK = 80
NBUF = 2
ESLICE = E // NS
LCH = ESLICE // CHUNK + 5
SBUF = 384
DUMMY_LOC = HALF

_SC_PARAMS = pltpu.CompilerParams(use_tc_tiling_on_sc=False,
                                  needs_layout_passes=False)


def _agg_pipeline(x_hbm, z_hbm, out_hbm, lpk_v, sstg_v, lstg_v, rows_v,
                  acc_sh, gsem, cid, sid, trips, zwaits):
    for w in zwaits:
        w()
    plsc.subcore_barrier()

    def stage_chunk(c, b):
        for k in range(CHUNK // 16):
            v = lpk_v[pl.ds(c * CHUNK + k * 16, 16)]
            sstg_v[pl.ds(b * CHUNK + k * 16, 16)] = jnp.bitwise_and(v, 0xFFFF)
            lstg_v[b, pl.ds(k * 16, 16)] = lax.shift_right_logical(v, 16)

    def src_idx(b):
        return sstg_v.at[pl.ds(b * CHUNK, CHUNK)]

    for b in range(NBUF):
        stage_chunk(b, b)
        pltpu.async_copy(x_hbm.at[src_idx(b)], rows_v.at[b], gsem)

    def step(i, carry):
        for b in range(NBUF):
            c = i * NBUF + b
            pltpu.make_async_copy(
                x_hbm.at[src_idx(b)], rows_v.at[b], gsem).wait()
            pltpu.sync_copy(rows_v.at[b], acc_sh.at[lstg_v.at[b]], add=True)
            stage_chunk(c + NBUF, b)

            @pl.when(c + NBUF < trips * NBUF)
            def _():
                pltpu.async_copy(x_hbm.at[src_idx(b)], rows_v.at[b], gsem)
        return carry

    lax.fori_loop(0, trips, step, 0)
    plsc.subcore_barrier()

    obase = sid * OUT_PER_TILE
    pltpu.sync_copy(acc_sh.at[pl.ds(obase, OUT_PER_TILE)],
                    out_hbm.at[cid, pl.ds(obase, OUT_PER_TILE)])


def _zero_starts(z_hbm, acc_sh, sid, gsem):
    zbase = sid * OUT_PER_TILE
    waits = []
    for j in range(OUT_PER_TILE // ZROWS):
        dst = acc_sh.at[pl.ds(zbase + j * ZROWS, ZROWS)]
        pltpu.async_copy(z_hbm, dst, gsem)
        waits.append(
            functools.partial(
                lambda d: pltpu.make_async_copy(z_hbm, d, gsem).wait(), dst))
    return waits


def _sc_aggregate_build(x, src2, dst2, zeros_blk):
    mesh = plsc.VectorSubcoreMesh(core_axis_name="c", subcore_axis_name="s")

    @functools.partial(
        pl.kernel,
        out_type=(jax.ShapeDtypeStruct((NC, HALF, D), jnp.float32),
                  jax.ShapeDtypeStruct((NC, NS, LCH * CHUNK), jnp.int32),
                  jax.ShapeDtypeStruct((NC, NS, 16), jnp.int32)),
        mesh=mesh,
        scratch_types=[
            pltpu.VMEM((ESLICE,), jnp.int32),
            pltpu.VMEM((ESLICE,), jnp.int32),
            pltpu.VMEM((LCH * CHUNK,), jnp.int32),
            pltpu.VMEM((NBUF * CHUNK,), jnp.int32),
            pltpu.VMEM((NBUF, CHUNK), jnp.int32),
            pltpu.VMEM((SBUF,), jnp.int32),
            pltpu.VMEM((16,), jnp.int32),
            pltpu.VMEM((NBUF, CHUNK, D), jnp.float32),
            pltpu.VMEM_SHARED((ACC_ROWS, D), jnp.float32),
            pltpu.SemaphoreType.DMA,
        ],
        compiler_params=_SC_PARAMS,
    )
    def agg_kernel(x_hbm, src_hbm, dst_hbm, z_hbm, out_hbm, lists_hbm,
                   counts_hbm, srcs_v, dsts_v, lpk_v, sstg_v, lstg_v, sbp_v,
                   cnt_v, rows_v, acc_sh, gsem):
        cid = lax.axis_index("c")
        sid = lax.axis_index("s")
        lo = cid * HALF

        pltpu.sync_copy(src_hbm.at[sid], srcs_v)
        pltpu.sync_copy(dst_hbm.at[sid], dsts_v)

        zwaits = _zero_starts(z_hbm, acc_sh, sid, gsem)

        dummy_pk = jnp.full((16,), DUMMY_LOC << 16, jnp.int32)

        def flush_row(r):
            for k in range(CHUNK // 16):
                v = sbp_v[pl.ds(k * 16, 16)]
                lpk_v[pl.ds(r * CHUNK + k * 16, 16)] = v
            sbp_v[pl.ds(0, 16)] = sbp_v[pl.ds(CHUNK, 16)]

        def scan_step(i4, carry):
            p, r = carry
            for u in range(4):
                i = i4 * 4 + u
                srcv = srcs_v[pl.ds(i * 16, 16)]
                dstv = dsts_v[pl.ds(i * 16, 16)]
                locv = dstv - lo
                m = jnp.logical_and(locv >= 0, locv < HALF)
                pkv = jnp.bitwise_or(srcv, lax.shift_left(locv, 16))
                keyv = jnp.where(m, 0, 1)
                _, pk_s = plsc.sort_key_val(keyv, pkv)
                sbp_v[pl.ds(p, 16)] = pk_s
                p2 = p + jnp.sum(jnp.where(m, 1, 0), axis=0)

                @pl.when(p2 >= CHUNK)
                def _(r=r):
                    flush_row(r)

                full = (p2 >= CHUNK).astype(jnp.int32)
                p, r = p2 - full * CHUNK, r + full
            return p, r

        p, r = lax.fori_loop(0, ESLICE // 64, scan_step, (0, 0))

        for k in range(CHUNK // 16 + 1):
            sbp_v[pl.ds(p + k * 16, 16)] = dummy_pk
        flush_row(r)
        for j in range(1, 4):
            for k in range(CHUNK // 16):
                lpk_v[pl.ds((r + j) * CHUNK + k * 16, 16)] = dummy_pk
        cnt = r * CHUNK + p
        nch = (cnt + CHUNK - 1) // CHUNK
        trips = jnp.maximum((nch + NBUF - 1) // NBUF, 1)

        pltpu.sync_copy(lpk_v, lists_hbm.at[cid, sid])
        cnt_v[pl.ds(0, 16)] = jnp.full((16,), cnt, jnp.int32)
        pltpu.sync_copy(cnt_v, counts_hbm.at[cid, sid])

        _agg_pipeline(x_hbm, z_hbm, out_hbm, lpk_v, sstg_v, lstg_v, rows_v,
                      acc_sh, gsem, cid, sid, trips, zwaits)

    return agg_kernel(x, src2, dst2, zeros_blk)


def _sc_aggregate_reuse(x, lists, counts, zeros_blk):
    mesh = plsc.VectorSubcoreMesh(core_axis_name="c", subcore_axis_name="s")

    @functools.partial(
        pl.kernel,
        out_type=jax.ShapeDtypeStruct((NC, HALF, D), jnp.float32),
        mesh=mesh,
        scratch_types=[
            pltpu.VMEM((LCH * CHUNK,), jnp.int32),
            pltpu.VMEM((NBUF * CHUNK,), jnp.int32),
            pltpu.VMEM((NBUF, CHUNK), jnp.int32),
            pltpu.VMEM((16,), jnp.int32),
            pltpu.VMEM((NBUF, CHUNK, D), jnp.float32),
            pltpu.VMEM_SHARED((ACC_ROWS, D), jnp.float32),
            pltpu.SemaphoreType.DMA,
        ],
        compiler_params=_SC_PARAMS,
    )
    def agg_kernel(x_hbm, lists_hbm, counts_hbm, z_hbm, out_hbm,
                   lpk_v, sstg_v, lstg_v, cnt_v, rows_v, acc_sh, gsem):
        cid = lax.axis_index("c")
        sid = lax.axis_index("s")

        zwaits = _zero_starts(z_hbm, acc_sh, sid, gsem)
        pltpu.sync_copy(lists_hbm.at[cid, sid], lpk_v)
        pltpu.sync_copy(counts_hbm.at[cid, sid], cnt_v)
        cnt = jnp.max(cnt_v[...], axis=0)
        nch = (cnt + CHUNK - 1) // CHUNK
        trips = jnp.maximum((nch + NBUF - 1) // NBUF, 1)

        _agg_pipeline(x_hbm, z_hbm, out_hbm, lpk_v, sstg_v, lstg_v, rows_v,
                      acc_sh, gsem, cid, sid, trips, zwaits)

    return agg_kernel(x, lists, counts, zeros_blk)


def _tc_mlp(x, agg, Wa, ba, Wb, bb, relu_out):
    BN = 2000
    grid = (N // BN,)

    def body(x_ref, a_ref, wa_ref, ba_ref, wb_ref, bb_ref, o_ref):
        rst = x_ref[...] + a_ref[...]
        hid = jnp.dot(rst, wa_ref[...], preferred_element_type=jnp.float32)
        hid = jnp.maximum(hid + ba_ref[...], 0.0)
        out = jnp.dot(hid, wb_ref[...], preferred_element_type=jnp.float32)
        out = out + bb_ref[...]
        if relu_out:
            out = jnp.maximum(out, 0.0)
        o_ref[...] = out

    row_spec = pl.BlockSpec((BN, D), lambda i: (i, 0))
    full_spec = pl.BlockSpec((D, D), lambda i: (0, 0))
    vec_spec = pl.BlockSpec((1, D), lambda i: (0, 0))
    return pl.pallas_call(
        body,
        grid=grid,
        in_specs=[row_spec, row_spec,
                  full_spec, vec_spec, full_spec, vec_spec],
        out_specs=row_spec,
        out_shape=jax.ShapeDtypeStruct((N, D), jnp.float32),
    )(x, agg, Wa, ba.reshape(1, D), Wb, bb.reshape(1, D))


def kernel(h, edge_index, W0a, b0a, W0b, b0b, W1a, b1a, W1b, b1b):
    x0 = h.T

    src2 = edge_index[0].reshape(NS, ESLICE)
    dst2 = edge_index[1].reshape(NS, ESLICE)
    zeros_blk = jnp.zeros((ZROWS, D), jnp.float32)

    p0, lists, counts = _sc_aggregate_build(x0, src2, dst2, zeros_blk)
    x1 = _tc_mlp(x0, p0.reshape(NC * HALF, D), W0a, b0a, W0b, b0b,
                 relu_out=True)
    p1 = _sc_aggregate_reuse(x1, lists, counts, zeros_blk)
    out = _tc_mlp(x1, p1.reshape(NC * HALF, D), W1a, b1a, W1b, b1b,
                  relu_out=False)
    return out.T


# speedup vs baseline: 1.0044x; 1.0044x over previous
import functools

import jax
import jax.numpy as jnp
from jax import lax
from jax.experimental import pallas as pl
from jax.experimental.pallas import tpu as pltpu
from jax.experimental.pallas import tpu_sc as plsc

N = 10000
D = 128
E = 320000

NC = 2
NS = 16

HALF = 5120
ACC_ROWS = HALF + 8
OUT_PER_TILE = HALF // NS
ZROWS = 160

CHUNK = 80
NBUF = 2
ESLICE = E // NS
LCH = ESLICE // CHUNK + 5
SBUF = 384
DUMMY_LOC = HALF

_SC_PARAMS = pltpu.CompilerParams(use_tc_tiling_on_sc=False,
                                  needs_layout_passes=False)


def _agg_pipeline(x_hbm, z_hbm, out_hbm, lpk_v, sstg_v, lstg_v, rows_v,
                  acc_sh, gsem, cid, sid, trips, zwaits):
    for w in zwaits:
        w()
    plsc.subcore_barrier()

    def stage_chunk(c, b):
        for k in range(CHUNK // 16):
            v = lpk_v[pl.ds(c * CHUNK + k * 16, 16)]
            sstg_v[pl.ds(b * CHUNK + k * 16, 16)] = jnp.bitwise_and(v, 0xFFFF)
            lstg_v[b, pl.ds(k * 16, 16)] = lax.shift_right_logical(v, 16)

    def src_idx(b):
        return sstg_v.at[pl.ds(b * CHUNK, CHUNK)]

    for b in range(NBUF):
        stage_chunk(b, b)
        pltpu.async_copy(x_hbm.at[src_idx(b)], rows_v.at[b], gsem)

    def step(i, carry):
        for b in range(NBUF):
            c = i * NBUF + b
            pltpu.make_async_copy(
                x_hbm.at[src_idx(b)], rows_v.at[b], gsem).wait()
            pltpu.sync_copy(rows_v.at[b], acc_sh.at[lstg_v.at[b]], add=True)
            stage_chunk(c + NBUF, b)

            @pl.when(c + NBUF < trips * NBUF)
            def _():
                pltpu.async_copy(x_hbm.at[src_idx(b)], rows_v.at[b], gsem)
        return carry

    lax.fori_loop(0, trips, step, 0)
    plsc.subcore_barrier()

    obase = sid * OUT_PER_TILE
    pltpu.sync_copy(acc_sh.at[pl.ds(obase, OUT_PER_TILE)],
                    out_hbm.at[cid, pl.ds(obase, OUT_PER_TILE)])


def _zero_starts(z_hbm, acc_sh, sid, gsem):
    zbase = sid * OUT_PER_TILE
    waits = []
    for j in range(OUT_PER_TILE // ZROWS):
        dst = acc_sh.at[pl.ds(zbase + j * ZROWS, ZROWS)]
        pltpu.async_copy(z_hbm, dst, gsem)
        waits.append(
            functools.partial(
                lambda d: pltpu.make_async_copy(z_hbm, d, gsem).wait(), dst))
    return waits


def _sc_aggregate_build(x, src2, dst2, zeros_blk):
    mesh = plsc.VectorSubcoreMesh(core_axis_name="c", subcore_axis_name="s")

    @functools.partial(
        pl.kernel,
        out_type=(jax.ShapeDtypeStruct((NC, HALF, D), jnp.float32),
                  jax.ShapeDtypeStruct((NC, NS, LCH * CHUNK), jnp.int32),
                  jax.ShapeDtypeStruct((NC, NS, 16), jnp.int32)),
        mesh=mesh,
        scratch_types=[
            pltpu.VMEM((ESLICE,), jnp.int32),
            pltpu.VMEM((ESLICE,), jnp.int32),
            pltpu.VMEM((LCH * CHUNK,), jnp.int32),
            pltpu.VMEM((NBUF * CHUNK,), jnp.int32),
            pltpu.VMEM((NBUF, CHUNK), jnp.int32),
            pltpu.VMEM((SBUF,), jnp.int32),
            pltpu.VMEM((16,), jnp.int32),
            pltpu.VMEM((NBUF, CHUNK, D), jnp.float32),
            pltpu.VMEM_SHARED((ACC_ROWS, D), jnp.float32),
            pltpu.SemaphoreType.DMA,
        ],
        compiler_params=_SC_PARAMS,
    )
    def agg_kernel(x_hbm, src_hbm, dst_hbm, z_hbm, out_hbm, lists_hbm,
                   counts_hbm, srcs_v, dsts_v, lpk_v, sstg_v, lstg_v, sbp_v,
                   cnt_v, rows_v, acc_sh, gsem):
        cid = lax.axis_index("c")
        sid = lax.axis_index("s")
        lo = cid * HALF

        pltpu.sync_copy(src_hbm.at[sid], srcs_v)
        pltpu.sync_copy(dst_hbm.at[sid], dsts_v)

        zwaits = _zero_starts(z_hbm, acc_sh, sid, gsem)

        dummy_pk = jnp.full((16,), DUMMY_LOC << 16, jnp.int32)

        def flush_row(r):
            for k in range(CHUNK // 16):
                v = sbp_v[pl.ds(k * 16, 16)]
                lpk_v[pl.ds(r * CHUNK + k * 16, 16)] = v
            sbp_v[pl.ds(0, 16)] = sbp_v[pl.ds(CHUNK, 16)]

        def scan_step(i, carry):
            p, r = carry
            srcv = srcs_v[pl.ds(i * 16, 16)]
            dstv = dsts_v[pl.ds(i * 16, 16)]
            locv = dstv - lo
            m = jnp.logical_and(locv >= 0, locv < HALF)
            pkv = jnp.bitwise_or(srcv, lax.shift_left(locv, 16))
            keyv = jnp.where(m, 0, 1)
            _, pk_s = plsc.sort_key_val(keyv, pkv)
            sbp_v[pl.ds(p, 16)] = pk_s
            p2 = p + jnp.sum(jnp.where(m, 1, 0), axis=0)

            @pl.when(p2 >= CHUNK)
            def _():
                flush_row(r)

            full = (p2 >= CHUNK).astype(jnp.int32)
            return p2 - full * CHUNK, r + full

        p, r = lax.fori_loop(0, ESLICE // 16, scan_step, (0, 0))

        for k in range(CHUNK // 16 + 1):
            sbp_v[pl.ds(p + k * 16, 16)] = dummy_pk
        flush_row(r)
        for j in range(1, 4):
            for k in range(CHUNK // 16):
                lpk_v[pl.ds((r + j) * CHUNK + k * 16, 16)] = dummy_pk
        cnt = r * CHUNK + p
        nch = (cnt + CHUNK - 1) // CHUNK
        trips = jnp.maximum((nch + NBUF - 1) // NBUF, 1)

        pltpu.sync_copy(lpk_v, lists_hbm.at[cid, sid])
        cnt_v[pl.ds(0, 16)] = jnp.full((16,), cnt, jnp.int32)
        pltpu.sync_copy(cnt_v, counts_hbm.at[cid, sid])

        _agg_pipeline(x_hbm, z_hbm, out_hbm, lpk_v, sstg_v, lstg_v, rows_v,
                      acc_sh, gsem, cid, sid, trips, zwaits)

    return agg_kernel(x, src2, dst2, zeros_blk)


def _sc_aggregate_reuse(x, lists, counts, zeros_blk):
    mesh = plsc.VectorSubcoreMesh(core_axis_name="c", subcore_axis_name="s")

    @functools.partial(
        pl.kernel,
        out_type=jax.ShapeDtypeStruct((NC, HALF, D), jnp.float32),
        mesh=mesh,
        scratch_types=[
            pltpu.VMEM((LCH * CHUNK,), jnp.int32),
            pltpu.VMEM((NBUF * CHUNK,), jnp.int32),
            pltpu.VMEM((NBUF, CHUNK), jnp.int32),
            pltpu.VMEM((16,), jnp.int32),
            pltpu.VMEM((NBUF, CHUNK, D), jnp.float32),
            pltpu.VMEM_SHARED((ACC_ROWS, D), jnp.float32),
            pltpu.SemaphoreType.DMA,
        ],
        compiler_params=_SC_PARAMS,
    )
    def agg_kernel(x_hbm, lists_hbm, counts_hbm, z_hbm, out_hbm,
                   lpk_v, sstg_v, lstg_v, cnt_v, rows_v, acc_sh, gsem):
        cid = lax.axis_index("c")
        sid = lax.axis_index("s")

        zwaits = _zero_starts(z_hbm, acc_sh, sid, gsem)
        pltpu.sync_copy(lists_hbm.at[cid, sid], lpk_v)
        pltpu.sync_copy(counts_hbm.at[cid, sid], cnt_v)
        cnt = jnp.max(cnt_v[...], axis=0)
        nch = (cnt + CHUNK - 1) // CHUNK
        trips = jnp.maximum((nch + NBUF - 1) // NBUF, 1)

        _agg_pipeline(x_hbm, z_hbm, out_hbm, lpk_v, sstg_v, lstg_v, rows_v,
                      acc_sh, gsem, cid, sid, trips, zwaits)

    return agg_kernel(x, lists, counts, zeros_blk)


def _tc_mlp(x, agg, Wa, ba, Wb, bb, relu_out):
    BN = 2000
    grid = (N // BN,)

    def body(x_ref, a_ref, wa_ref, ba_ref, wb_ref, bb_ref, o_ref):
        rst = x_ref[...] + a_ref[...]
        hid = jnp.dot(rst, wa_ref[...], preferred_element_type=jnp.float32)
        hid = jnp.maximum(hid + ba_ref[...], 0.0)
        out = jnp.dot(hid, wb_ref[...], preferred_element_type=jnp.float32)
        out = out + bb_ref[...]
        if relu_out:
            out = jnp.maximum(out, 0.0)
        o_ref[...] = out

    row_spec = pl.BlockSpec((BN, D), lambda i: (i, 0))
    full_spec = pl.BlockSpec((D, D), lambda i: (0, 0))
    vec_spec = pl.BlockSpec((1, D), lambda i: (0, 0))
    return pl.pallas_call(
        body,
        grid=grid,
        in_specs=[row_spec, row_spec,
                  full_spec, vec_spec, full_spec, vec_spec],
        out_specs=row_spec,
        out_shape=jax.ShapeDtypeStruct((N, D), jnp.float32),
    )(x, agg, Wa, ba.reshape(1, D), Wb, bb.reshape(1, D))


def kernel(h, edge_index, W0a, b0a, W0b, b0b, W1a, b1a, W1b, b1b):
    x0 = h.T

    src2 = edge_index[0].reshape(NS, ESLICE)
    dst2 = edge_index[1].reshape(NS, ESLICE)
    zeros_blk = jnp.zeros((ZROWS, D), jnp.float32)

    p0, lists, counts = _sc_aggregate_build(x0, src2, dst2, zeros_blk)
    x1 = _tc_mlp(x0, p0.reshape(NC * HALF, D), W0a, b0a, W0b, b0b,
                 relu_out=True)
    p1 = _sc_aggregate_reuse(x1, lists, counts, zeros_blk)
    out = _tc_mlp(x1, p1.reshape(NC * HALF, D), W1a, b1a, W1b, b1b,
                  relu_out=False)
    return out.T
